# Initial kernel scaffold; baseline (speedup 1.0000x reference)
import jax
import jax.numpy as jnp
from jax.experimental import pallas as pl


def kernel(x, edge_attr, edge_contact_attr, edge_index, edge_contact_index, params):
    raise NotImplementedError("write your pallas kernel here")


# speedup vs baseline: 1.7798x; 1.7798x over previous
import functools

import jax
import jax.numpy as jnp
from jax import lax
from jax.experimental import pallas as pl
from jax.experimental.pallas import tpu as pltpu
from jax.experimental.pallas import tpu_sc as plsc

N_NODES = 50000
HID = 64
E_MESH = 800000
E_CONT = 100000

NC = 2
NS = 16
NW = NC * NS

CH = 128
NHALF = N_NODES // NC
ACC_ROWS = NHALF + 8
DUMP = NHALF
WCH = 250
NZCH = NHALF // WCH


def _mlp_tc(inputs, w1s, b1, w2, b2, gamma, beta, rows, block, out_pad):
    nin = len(inputs)
    ln = gamma is not None
    b1 = b1.reshape(1, -1)
    b2 = b2.reshape(1, -1)
    in_specs = []
    for arr, off in inputs:
        ob = off // block
        in_specs.append(
            pl.BlockSpec((block, arr.shape[1]), lambda i, ob=ob: (i + ob, 0)))
    for w in w1s:
        in_specs.append(pl.BlockSpec(w.shape, lambda i: (0, 0)))
    for v in (b1, w2, b2):
        in_specs.append(pl.BlockSpec(v.shape, lambda i: (0, 0)))
    extra = []
    if ln:
        gamma = gamma.reshape(1, -1)
        beta = beta.reshape(1, -1)
        extra = [gamma, beta]
        for v in extra:
            in_specs.append(pl.BlockSpec(v.shape, lambda i: (0, 0)))
    out_dim = w2.shape[1]

    def body(*refs):
        xr = refs[:nin]
        wr = refs[nin:2 * nin]
        b1r, w2r, b2r = refs[2 * nin:2 * nin + 3]
        o = refs[-1]
        acc = jnp.dot(xr[0][...], wr[0][...], preferred_element_type=jnp.float32)
        for k in range(1, nin):
            acc = acc + jnp.dot(xr[k][...], wr[k][...],
                                preferred_element_type=jnp.float32)
        h = jnp.maximum(acc + b1r[...], 0.0)
        y = jnp.dot(h, w2r[...], preferred_element_type=jnp.float32) + b2r[...]
        if ln:
            gr, br = refs[2 * nin + 3], refs[2 * nin + 4]
            mu = jnp.mean(y, axis=-1, keepdims=True)
            var = jnp.mean((y - mu) ** 2, axis=-1, keepdims=True)
            y = (y - mu) * lax.rsqrt(var + 1e-5) * gr[...] + br[...]
        o[...] = y

    args = [a for a, _ in inputs] + list(w1s) + [b1, w2, b2] + extra
    return pl.pallas_call(
        body,
        grid=(rows // block,),
        in_specs=in_specs,
        out_specs=pl.BlockSpec((block, out_dim), lambda i: (i, 0)),
        out_shape=jax.ShapeDtypeStruct((out_pad, out_dim), jnp.float32),
    )(*args)


def _node_update_tc(node, summ, cntm, sumc, cntc, p, block=2000):
    w1 = p["W1"]
    w1a, w1b, w1c = w1[0:HID], w1[HID:2 * HID], w1[2 * HID:3 * HID]
    b1 = p["b1"].reshape(1, -1)
    b2 = p["b2"].reshape(1, -1)
    g = p["g"].reshape(1, -1)
    beta = p["beta"].reshape(1, -1)

    def body(nr, smr, cmr, scr, ccr, w1ar, w1br, w1cr, b1r, w2r, b2r, gr, br, o):
        rm = 1.0 / jnp.maximum(cmr[...][:, 0:1], 1.0)
        rc = 1.0 / jnp.maximum(ccr[...][:, 0:1], 1.0)
        aggm = smr[...] * rm
        aggc = scr[...] * rc
        acc = (jnp.dot(nr[...], w1ar[...], preferred_element_type=jnp.float32)
               + jnp.dot(aggm, w1br[...], preferred_element_type=jnp.float32)
               + jnp.dot(aggc, w1cr[...], preferred_element_type=jnp.float32))
        h = jnp.maximum(acc + b1r[...], 0.0)
        y = jnp.dot(h, w2r[...], preferred_element_type=jnp.float32) + b2r[...]
        mu = jnp.mean(y, axis=-1, keepdims=True)
        var = jnp.mean((y - mu) ** 2, axis=-1, keepdims=True)
        o[...] = (y - mu) * lax.rsqrt(var + 1e-5) * gr[...] + br[...]

    full = lambda a: pl.BlockSpec(a.shape, lambda i: (0, 0))
    rowspec = lambda d: pl.BlockSpec((block, d), lambda i: (i, 0))
    return pl.pallas_call(
        body,
        grid=(N_NODES // block,),
        in_specs=[rowspec(HID), rowspec(HID), rowspec(16), rowspec(HID),
                  rowspec(16), full(w1a), full(w1b), full(w1c), full(b1),
                  full(p["W2"]), full(b2), full(g), full(beta)],
        out_specs=rowspec(HID),
        out_shape=jax.ShapeDtypeStruct((N_NODES, HID), jnp.float32),
    )(node, summ, cntm, sumc, cntc, w1a, w1b, w1c, b1, p["W2"], b2, g, beta)


def _sc_gather(table, idx):
    total = idx.shape[0]
    per_w = total // NW
    n_ch = per_w // CH
    mesh = plsc.VectorSubcoreMesh(core_axis_name="c", subcore_axis_name="s")

    @functools.partial(
        pl.kernel, mesh=mesh,
        out_type=jax.ShapeDtypeStruct((total, HID), jnp.float32),
        compiler_params=pltpu.CompilerParams(use_tc_tiling_on_sc=False),
        scratch_types=[
            pltpu.VMEM((CH,), jnp.int32),
            pltpu.VMEM((CH, HID), jnp.float32),
            pltpu.SemaphoreType.DMA,
        ])
    def k(table_hbm, idx_hbm, out_hbm, idx_v, rows_v, sem):
        c = lax.axis_index("c")
        s = lax.axis_index("s")
        wid = s * NC + c
        base = wid * per_w

        def body(i, carry):
            off = base + i * CH
            pltpu.sync_copy(idx_hbm.at[pl.ds(off, CH)], idx_v)
            pltpu.async_copy(table_hbm.at[idx_v], rows_v, sem).wait()
            pltpu.sync_copy(rows_v, out_hbm.at[pl.ds(off, CH)])
            return carry

        lax.fori_loop(0, n_ch, body, 0)

    return k(table, idx)


def _sc_scatter_sum(vals, dst, d):
    e_pad = dst.shape[0]
    per_t = e_pad // NS
    n_ch = per_t // CH
    count_mode = vals is None
    mesh = plsc.VectorSubcoreMesh(core_axis_name="c", subcore_axis_name="s")

    def body(*refs):
        if count_mode:
            dst_hbm, out_hbm, idx_v, li_v, rows_v, tmp_v, acc = refs
            vals_hbm = None
        else:
            vals_hbm, dst_hbm, out_hbm, idx_v, li_v, rows_v, tmp_v, acc = refs
        c = lax.axis_index("c")
        s = lax.axis_index("s")
        nbase = c * NHALF
        zero16 = (lax.iota(jnp.int32, 16) * 0).astype(jnp.float32)
        one16 = zero16 + 1.0

        def zrow(i, carry):
            for j in range(d // 16):
                tmp_v[i, pl.ds(j * 16, 16)] = zero16
            return carry
        lax.fori_loop(0, WCH, zrow, 0)
        nz = (NZCH - 1 - s) // NS + 1

        def zchunk(k2, carry):
            start = (s + k2 * NS) * WCH
            pltpu.sync_copy(tmp_v, acc.at[pl.ds(start, WCH)])
            return carry
        lax.fori_loop(0, nz, zchunk, 0)
        plsc.subcore_barrier()

        if count_mode:
            def orow(i, carry):
                for j in range(d // 16):
                    rows_v[i, pl.ds(j * 16, 16)] = one16
                return carry
            lax.fori_loop(0, CH, orow, 0)

        def sc_body(i, carry):
            off = s * per_t + i * CH
            pltpu.sync_copy(dst_hbm.at[pl.ds(off, CH)], idx_v)
            for j in range(CH // 16):
                v = idx_v[pl.ds(j * 16, 16)]
                li = v - nbase
                ok = (li >= 0) & (li < NHALF)
                li_v[pl.ds(j * 16, 16)] = jnp.where(ok, li, DUMP)
            if not count_mode:
                pltpu.sync_copy(vals_hbm.at[pl.ds(off, CH)], rows_v)
            pltpu.sync_copy(rows_v, acc.at[li_v], add=True)
            return carry
        lax.fori_loop(0, n_ch, sc_body, 0)
        plsc.subcore_barrier()

        def wchunk(k2, carry):
            start = (s + k2 * NS) * WCH
            pltpu.sync_copy(acc.at[pl.ds(start, WCH)], tmp_v)
            pltpu.sync_copy(tmp_v, out_hbm.at[pl.ds(nbase + start, WCH)])
            return carry
        lax.fori_loop(0, nz, wchunk, 0)

    scratch = [
        pltpu.VMEM((CH,), jnp.int32),
        pltpu.VMEM((CH,), jnp.int32),
        pltpu.VMEM((CH, d), jnp.float32),
        pltpu.VMEM((WCH, d), jnp.float32),
        pltpu.VMEM_SHARED((ACC_ROWS, d), jnp.float32),
    ]
    args = [dst] if count_mode else [vals, dst]
    return pl.kernel(
        body, mesh=mesh,
        out_type=jax.ShapeDtypeStruct((N_NODES, d), jnp.float32),
        compiler_params=pltpu.CompilerParams(use_tc_tiling_on_sc=False),
        scratch_types=scratch)(*args)


def _pad_rows(a, n):
    return jnp.pad(a, ((0, n - a.shape[0]),) + ((0, 0),) * (a.ndim - 1))


def _pad_k(w):
    return jnp.pad(w, ((0, 8 - w.shape[0]), (0, 0)))


def kernel(x, edge_attr, edge_contact_attr, edge_index, edge_contact_index, params):
    p = params
    s, r = edge_index[0], edge_index[1]
    cs, cr = edge_contact_index[0], edge_contact_index[1]

    em_pad = ((E_MESH + NW * CH - 1) // (NW * CH)) * NW * CH
    ec_pad = ((E_CONT + NW * CH - 1) // (NW * CH)) * NW * CH
    g_rows = E_MESH * 2 + E_CONT * 2
    g_pad = ((g_rows + NW * CH - 1) // (NW * CH)) * NW * CH

    r_pad = jnp.pad(r, (0, em_pad - E_MESH), constant_values=-1)
    cr_pad = jnp.pad(cr, (0, ec_pad - E_CONT), constant_values=-1)
    idx_all = jnp.concatenate(
        [s, r, cs, cr, jnp.zeros((g_pad - g_rows,), jnp.int32)])

    x8 = jnp.pad(x, ((0, 0), (0, 8 - x.shape[1])))
    ea8 = jnp.pad(edge_attr, ((0, 0), (0, 8 - edge_attr.shape[1])))
    ca8 = jnp.pad(edge_contact_attr, ((0, 0), (0, 8 - edge_contact_attr.shape[1])))

    ne = p["node_enc"]
    node = _mlp_tc([(x8, 0)], [_pad_k(ne["W1"])], ne["b1"], ne["W2"], ne["b2"],
                   ne["g"], ne["beta"], N_NODES, 2000, N_NODES)
    ee = p["edge_enc"]
    ea = _mlp_tc([(ea8, 0)], [_pad_k(ee["W1"])], ee["b1"], ee["W2"], ee["b2"],
                 ee["g"], ee["beta"], E_MESH, 6400, E_MESH)
    ce = p["cont_enc"]
    eca = _mlp_tc([(ca8, 0)], [_pad_k(ce["W1"])], ce["b1"], ce["W2"], ce["b2"],
                  ce["g"], ce["beta"], E_CONT, 4000, E_CONT)

    cntm = _sc_scatter_sum(None, r_pad, 16)
    cntc = _sc_scatter_sum(None, cr_pad, 16)

    for ps in p["steps"]:
        g = _sc_gather(node, idx_all)
        em = ps["edge_mesh"]
        w1 = em["W1"]
        ea = _mlp_tc([(g, 0), (g, E_MESH), (ea, 0)],
                     [w1[0:HID], w1[HID:2 * HID], w1[2 * HID:3 * HID]],
                     em["b1"], em["W2"], em["b2"], em["g"], em["beta"],
                     E_MESH, 6400, em_pad)
        ecp = ps["edge_cont"]
        w1 = ecp["W1"]
        eca = _mlp_tc([(g, 2 * E_MESH), (g, 2 * E_MESH + E_CONT), (eca, 0)],
                      [w1[0:HID], w1[HID:2 * HID], w1[2 * HID:3 * HID]],
                      ecp["b1"], ecp["W2"], ecp["b2"], ecp["g"], ecp["beta"],
                      E_CONT, 4000, ec_pad)
        summ = _sc_scatter_sum(ea, r_pad, HID)
        sumc = _sc_scatter_sum(eca, cr_pad, HID)
        node = _node_update_tc(node, summ, cntm, sumc, cntc, ps["node"])

    de = p["dec"]
    out = _mlp_tc([(node, 0)], [de["W1"]], de["b1"], de["W2"], de["b2"],
                  None, None, N_NODES, 2000, N_NODES)
    return out


# speedup vs baseline: 2.1350x; 1.1996x over previous
import functools

import jax
import jax.numpy as jnp
from jax import lax
from jax.experimental import pallas as pl
from jax.experimental.pallas import tpu as pltpu
from jax.experimental.pallas import tpu_sc as plsc

N_NODES = 50000
HID = 64
E_MESH = 800000
E_CONT = 100000

NC = 2
NS = 16
NW = NC * NS

SUB = 128
GK = 4
GRP = SUB * GK
GRP_S = SUB
NHALF = N_NODES // NC
ACC_ROWS = NHALF + 8
DUMP = NHALF
WCH = 100
NZCH = NHALF // WCH


def _mlp_tc(inputs, w1s, b1, w2, b2, gamma, beta, rows, block, out_pad):
    nin = len(inputs)
    ln = gamma is not None
    b1 = b1.reshape(1, -1)
    b2 = b2.reshape(1, -1)
    in_specs = []
    for arr, off in inputs:
        ob = off // block
        in_specs.append(
            pl.BlockSpec((block, arr.shape[1]), lambda i, ob=ob: (i + ob, 0)))
    for w in w1s:
        in_specs.append(pl.BlockSpec(w.shape, lambda i: (0, 0)))
    for v in (b1, w2, b2):
        in_specs.append(pl.BlockSpec(v.shape, lambda i: (0, 0)))
    extra = []
    if ln:
        gamma = gamma.reshape(1, -1)
        beta = beta.reshape(1, -1)
        extra = [gamma, beta]
        for v in extra:
            in_specs.append(pl.BlockSpec(v.shape, lambda i: (0, 0)))
    out_dim = w2.shape[1]

    def body(*refs):
        xr = refs[:nin]
        wr = refs[nin:2 * nin]
        b1r, w2r, b2r = refs[2 * nin:2 * nin + 3]
        o = refs[-1]
        acc = jnp.dot(xr[0][...], wr[0][...], preferred_element_type=jnp.float32)
        for k in range(1, nin):
            acc = acc + jnp.dot(xr[k][...], wr[k][...],
                                preferred_element_type=jnp.float32)
        h = jnp.maximum(acc + b1r[...], 0.0)
        y = jnp.dot(h, w2r[...], preferred_element_type=jnp.float32) + b2r[...]
        if ln:
            gr, br = refs[2 * nin + 3], refs[2 * nin + 4]
            mu = jnp.mean(y, axis=-1, keepdims=True)
            var = jnp.mean((y - mu) ** 2, axis=-1, keepdims=True)
            y = (y - mu) * lax.rsqrt(var + 1e-5) * gr[...] + br[...]
        o[...] = y

    args = [a for a, _ in inputs] + list(w1s) + [b1, w2, b2] + extra
    return pl.pallas_call(
        body,
        grid=(rows // block,),
        in_specs=in_specs,
        out_specs=pl.BlockSpec((block, out_dim), lambda i: (i, 0)),
        out_shape=jax.ShapeDtypeStruct((out_pad, out_dim), jnp.float32),
    )(*args)


def _node_update_tc(node, summ, cntm, sumc, cntc, p, block=2000):
    w1 = p["W1"]
    w1a, w1b, w1c = w1[0:HID], w1[HID:2 * HID], w1[2 * HID:3 * HID]
    b1 = p["b1"].reshape(1, -1)
    b2 = p["b2"].reshape(1, -1)
    g = p["g"].reshape(1, -1)
    beta = p["beta"].reshape(1, -1)

    def body(nr, smr, cmr, scr, ccr, w1ar, w1br, w1cr, b1r, w2r, b2r, gr, br, o):
        rm = 1.0 / jnp.maximum(cmr[...][:, 0:1], 1.0)
        rc = 1.0 / jnp.maximum(ccr[...][:, 0:1], 1.0)
        aggm = smr[...] * rm
        aggc = scr[...] * rc
        acc = (jnp.dot(nr[...], w1ar[...], preferred_element_type=jnp.float32)
               + jnp.dot(aggm, w1br[...], preferred_element_type=jnp.float32)
               + jnp.dot(aggc, w1cr[...], preferred_element_type=jnp.float32))
        h = jnp.maximum(acc + b1r[...], 0.0)
        y = jnp.dot(h, w2r[...], preferred_element_type=jnp.float32) + b2r[...]
        mu = jnp.mean(y, axis=-1, keepdims=True)
        var = jnp.mean((y - mu) ** 2, axis=-1, keepdims=True)
        o[...] = (y - mu) * lax.rsqrt(var + 1e-5) * gr[...] + br[...]

    full = lambda a: pl.BlockSpec(a.shape, lambda i: (0, 0))
    rowspec = lambda d: pl.BlockSpec((block, d), lambda i: (i, 0))
    return pl.pallas_call(
        body,
        grid=(N_NODES // block,),
        in_specs=[rowspec(HID), rowspec(HID), rowspec(16), rowspec(HID),
                  rowspec(16), full(w1a), full(w1b), full(w1c), full(b1),
                  full(p["W2"]), full(b2), full(g), full(beta)],
        out_specs=rowspec(HID),
        out_shape=jax.ShapeDtypeStruct((N_NODES, HID), jnp.float32),
    )(node, summ, cntm, sumc, cntc, w1a, w1b, w1c, b1, p["W2"], b2, g, beta)


def _sc_gather(table, idx):
    total = idx.shape[0]
    per_w = total // NW
    n_g = per_w // GRP
    n_g2 = n_g // 2
    mesh = plsc.VectorSubcoreMesh(core_axis_name="c", subcore_axis_name="s")

    @functools.partial(
        pl.kernel, mesh=mesh,
        out_type=jax.ShapeDtypeStruct((total, HID), jnp.float32),
        compiler_params=pltpu.CompilerParams(use_tc_tiling_on_sc=False),
        scratch_types=[
            pltpu.VMEM((2, GRP), jnp.int32),
            pltpu.VMEM((2, GRP, HID), jnp.float32),
        ] + [pltpu.SemaphoreType.DMA] * 6)
    def k(table_hbm, idx_hbm, out_hbm, idx2, rows2, si0, si1, sg0, sg1, so0, so1):
        c = lax.axis_index("c")
        s = lax.axis_index("s")
        base = (s * NC + c) * per_w
        si = (si0, si1)
        sg = (sg0, sg1)
        so = (so0, so1)
        pltpu.async_copy(idx_hbm.at[pl.ds(base, GRP)], idx2.at[0], si0)
        pltpu.async_copy(idx_hbm.at[pl.ds(base + GRP, GRP)], idx2.at[1], si1)

        def g2_body(g2, carry):
            for par in range(2):
                g = g2 * 2 + par
                off = base + g * GRP
                pltpu.make_async_copy(
                    idx_hbm.at[pl.ds(off, GRP)], idx2.at[par], si[par]).wait()

                @pl.when(g2 > 0)
                def _():
                    pltpu.make_async_copy(
                        rows2.at[par], out_hbm.at[pl.ds(off, GRP)],
                        so[par]).wait()

                for b in range(GK):
                    pltpu.async_copy(
                        table_hbm.at[idx2.at[par, pl.ds(b * SUB, SUB)]],
                        rows2.at[par, pl.ds(b * SUB, SUB)], sg[par])
                for b in range(GK):
                    pltpu.make_async_copy(
                        table_hbm.at[idx2.at[par, pl.ds(0, SUB)]],
                        rows2.at[par, pl.ds(0, SUB)], sg[par]).wait()
                pltpu.async_copy(rows2.at[par], out_hbm.at[pl.ds(off, GRP)],
                                 so[par])

                @pl.when(g2 + 1 < n_g2)
                def _():
                    pltpu.async_copy(
                        idx_hbm.at[pl.ds(off + 2 * GRP, GRP)], idx2.at[par],
                        si[par])
            return carry

        lax.fori_loop(0, n_g2, g2_body, 0)
        pltpu.make_async_copy(rows2.at[0], out_hbm.at[pl.ds(base, GRP)],
                              so0).wait()
        pltpu.make_async_copy(rows2.at[1], out_hbm.at[pl.ds(base, GRP)],
                              so1).wait()

    return k(table, idx)


def _sc_scatter_sum(vals_list, dst_list, d):
    nph = len(dst_list)
    count_mode = vals_list[0] is None
    mesh = plsc.VectorSubcoreMesh(core_axis_name="c", subcore_axis_name="s")

    def body(*refs):
        nin = nph if count_mode else 2 * nph
        ins = refs[:nin]
        outs = refs[nin:nin + nph]
        idx2, li2, zw, acc, rows = refs[nin + nph:nin + nph + 5]
        sems = refs[nin + nph + 5:]
        si = sems[0:2]
        ssc = sems[2:4]
        sz = sems[4]
        sr = sems[5:7]
        c = lax.axis_index("c")
        s = lax.axis_index("s")
        nbase = c * NHALF
        zero16 = (lax.iota(jnp.int32, 16) * 0).astype(jnp.float32)
        nz = (NZCH - 1 - s) // NS + 1

        def zrow(i, carry):
            for j in range(d // 16):
                zw[0, i, pl.ds(j * 16, 16)] = zero16
            return carry
        lax.fori_loop(0, WCH, zrow, 0)
        if count_mode:
            one16 = zero16 + 1.0

            def orow(i, carry):
                for j in range(d // 16):
                    rows[i, pl.ds(j * 16, 16)] = one16
                return carry
            lax.fori_loop(0, GRP_S, orow, 0)

        for ph in range(nph):
            if count_mode:
                dst_hbm = ins[ph]
                vals_hbm = None
            else:
                vals_hbm, dst_hbm = ins[2 * ph], ins[2 * ph + 1]
            out_hbm = outs[ph]
            e_pad = dst_hbm.shape[0]
            per_t = e_pad // NS
            n_g2 = per_t // GRP_S // 2

            def zfire(k2, carry):
                start = (s + k2 * NS) * WCH
                pltpu.async_copy(zw.at[0], acc.at[pl.ds(start, WCH)], sz)
                return carry
            lax.fori_loop(0, nz, zfire, 0)

            def zwait(k2, carry):
                pltpu.make_async_copy(zw.at[0], acc.at[pl.ds(0, WCH)],
                                      sz).wait()
                return carry
            lax.fori_loop(0, nz, zwait, 0)
            plsc.subcore_barrier()

            tbase = s * per_t
            pltpu.async_copy(dst_hbm.at[pl.ds(tbase, GRP_S)], idx2.at[0],
                             si[0])
            pltpu.async_copy(dst_hbm.at[pl.ds(tbase + GRP_S, GRP_S)],
                             idx2.at[1], si[1])
            if not count_mode:
                pltpu.async_copy(vals_hbm.at[pl.ds(tbase, GRP_S)], rows.at[0],
                                 sr[0])
                pltpu.async_copy(vals_hbm.at[pl.ds(tbase + GRP_S, GRP_S)],
                                 rows.at[1], sr[1])

            def g2_body(g2, carry):
                for par in range(2):
                    off = tbase + (g2 * 2 + par) * GRP_S
                    pltpu.make_async_copy(
                        dst_hbm.at[pl.ds(off, GRP_S)], idx2.at[par],
                        si[par]).wait()
                    if not count_mode:
                        pltpu.make_async_copy(
                            vals_hbm.at[pl.ds(off, GRP_S)], rows.at[par],
                            sr[par]).wait()
                    for j in range(GRP_S // 16):
                        v = idx2[par, pl.ds(j * 16, 16)]
                        li = v - nbase
                        ok = (li >= 0) & (li < NHALF)
                        li2[par, 0, pl.ds(j * 16, 16)] = jnp.where(ok, li,
                                                                   DUMP)
                    src = rows if count_mode else rows.at[par]
                    pltpu.async_copy(src, acc.at[li2.at[par, 0]], ssc[par],
                                     add=True)
                    pltpu.make_async_copy(src, acc.at[li2.at[par, 0]],
                                          ssc[par]).wait()

                    @pl.when(g2 + 1 < n_g2)
                    def _():
                        pltpu.async_copy(
                            dst_hbm.at[pl.ds(off + 2 * GRP_S, GRP_S)],
                            idx2.at[par], si[par])
                        if not count_mode:
                            pltpu.async_copy(
                                vals_hbm.at[pl.ds(off + 2 * GRP_S, GRP_S)],
                                rows.at[par], sr[par])
                return carry

            lax.fori_loop(0, n_g2, g2_body, 0)
            plsc.subcore_barrier()

            def wchunk(k2, carry):
                start = (s + k2 * NS) * WCH
                pltpu.sync_copy(acc.at[pl.ds(start, WCH)], zw.at[1])
                pltpu.sync_copy(zw.at[1], out_hbm.at[pl.ds(nbase + start,
                                                           WCH)])
                return carry
            lax.fori_loop(0, nz, wchunk, 0)
            plsc.subcore_barrier()

    scratch = [
        pltpu.VMEM((2, GRP_S), jnp.int32),
        pltpu.VMEM((2, 1, GRP_S), jnp.int32),
        pltpu.VMEM((2, WCH, d), jnp.float32),
        pltpu.VMEM_SHARED((ACC_ROWS, d), jnp.float32),
        pltpu.VMEM((GRP_S, d), jnp.float32) if count_mode
        else pltpu.VMEM((2, GRP_S, d), jnp.float32),
    ]
    scratch += [pltpu.SemaphoreType.DMA] * 7
    args = []
    for ph in range(nph):
        if not count_mode:
            args.append(vals_list[ph])
        args.append(dst_list[ph])
    return pl.kernel(
        body, mesh=mesh,
        out_type=tuple(
            jax.ShapeDtypeStruct((N_NODES, d), jnp.float32)
            for _ in range(nph)),
        compiler_params=pltpu.CompilerParams(use_tc_tiling_on_sc=False),
        scratch_types=scratch)(*args)


def _pad_rows(a, n):
    return jnp.pad(a, ((0, n - a.shape[0]),) + ((0, 0),) * (a.ndim - 1))


def _pad_k(w):
    return jnp.pad(w, ((0, 8 - w.shape[0]), (0, 0)))


def kernel(x, edge_attr, edge_contact_attr, edge_index, edge_contact_index, params):
    p = params
    s, r = edge_index[0], edge_index[1]
    cs, cr = edge_contact_index[0], edge_contact_index[1]

    scat_unit = NS * GRP_S * 2
    gath_unit = NW * GRP * 2
    em_pad = ((E_MESH + scat_unit - 1) // scat_unit) * scat_unit
    ec_pad = ((E_CONT + scat_unit - 1) // scat_unit) * scat_unit
    g_rows = E_MESH * 2 + E_CONT * 2
    g_pad = ((g_rows + gath_unit - 1) // gath_unit) * gath_unit

    r_pad = jnp.pad(r, (0, em_pad - E_MESH), constant_values=-1)
    cr_pad = jnp.pad(cr, (0, ec_pad - E_CONT), constant_values=-1)
    idx_all = jnp.concatenate(
        [s, r, cs, cr, jnp.zeros((g_pad - g_rows,), jnp.int32)])

    x8 = jnp.pad(x, ((0, 0), (0, 8 - x.shape[1])))
    ea8 = jnp.pad(edge_attr, ((0, 0), (0, 8 - edge_attr.shape[1])))
    ca8 = jnp.pad(edge_contact_attr, ((0, 0), (0, 8 - edge_contact_attr.shape[1])))

    ne = p["node_enc"]
    node = _mlp_tc([(x8, 0)], [_pad_k(ne["W1"])], ne["b1"], ne["W2"], ne["b2"],
                   ne["g"], ne["beta"], N_NODES, 2000, N_NODES)
    ee = p["edge_enc"]
    ea = _mlp_tc([(ea8, 0)], [_pad_k(ee["W1"])], ee["b1"], ee["W2"], ee["b2"],
                 ee["g"], ee["beta"], E_MESH, 6400, E_MESH)
    ce = p["cont_enc"]
    eca = _mlp_tc([(ca8, 0)], [_pad_k(ce["W1"])], ce["b1"], ce["W2"], ce["b2"],
                  ce["g"], ce["beta"], E_CONT, 4000, E_CONT)

    cntm, cntc = _sc_scatter_sum([None, None], [r_pad, cr_pad], 16)

    for ps in p["steps"]:
        g = _sc_gather(node, idx_all)
        em = ps["edge_mesh"]
        w1 = em["W1"]
        ea = _mlp_tc([(g, 0), (g, E_MESH), (ea, 0)],
                     [w1[0:HID], w1[HID:2 * HID], w1[2 * HID:3 * HID]],
                     em["b1"], em["W2"], em["b2"], em["g"], em["beta"],
                     E_MESH, 6400, em_pad)
        ecp = ps["edge_cont"]
        w1 = ecp["W1"]
        eca = _mlp_tc([(g, 2 * E_MESH), (g, 2 * E_MESH + E_CONT), (eca, 0)],
                      [w1[0:HID], w1[HID:2 * HID], w1[2 * HID:3 * HID]],
                      ecp["b1"], ecp["W2"], ecp["b2"], ecp["g"], ecp["beta"],
                      E_CONT, 4000, ec_pad)
        summ, sumc = _sc_scatter_sum([ea, eca], [r_pad, cr_pad], HID)
        node = _node_update_tc(node, summ, cntm, sumc, cntc, ps["node"])

    de = p["dec"]
    out = _mlp_tc([(node, 0)], [de["W1"]], de["b1"], de["W2"], de["b2"],
                  None, None, N_NODES, 2000, N_NODES)
    return out


# speedup vs baseline: 2.7264x; 1.2770x over previous
import functools

import jax
import jax.numpy as jnp
from jax import lax
from jax.experimental import pallas as pl
from jax.experimental.pallas import tpu as pltpu
from jax.experimental.pallas import tpu_sc as plsc

N_NODES = 50000
HID = 64
E_MESH = 800000
E_CONT = 100000

NC = 2
NS = 16
NW = NC * NS

SUB = 128
GK = 4
GRP = SUB * GK
GRP_S = SUB
NHALF = N_NODES // NC
ACC_ROWS = NHALF + 8
DUMP = NHALF
WCH = 100
NZCH = NHALF // WCH


def _bd(w):
    z = jnp.zeros_like(w)
    return jnp.concatenate(
        [jnp.concatenate([w, z], axis=1), jnp.concatenate([z, w], axis=1)],
        axis=0)


def _t2(v):
    return jnp.concatenate([v, v]).reshape(1, -1)


def _ln_m():
    e = jnp.eye(2, dtype=jnp.float32)
    return jnp.kron(e, jnp.full((HID, HID), 1.0 / HID, jnp.float32))


def _mlp_tc(inputs, w1s, b1, w2, b2, gamma, beta, rows, block, out_pad):
    nin = len(inputs)
    ln = gamma is not None
    b1 = _t2(b1)
    b2 = _t2(b2)
    w1s = [_bd(w) for w in w1s]
    w2 = _bd(w2)
    in_specs = []
    for arr, off in inputs:
        ob = off // block
        in_specs.append(
            pl.BlockSpec((block, arr.shape[1]), lambda i, ob=ob: (i + ob, 0)))
    for w in w1s:
        in_specs.append(pl.BlockSpec(w.shape, lambda i: (0, 0)))
    for v in (b1, w2, b2):
        in_specs.append(pl.BlockSpec(v.shape, lambda i: (0, 0)))
    extra = []
    if ln:
        extra = [_t2(gamma), _t2(beta), _ln_m()]
        for v in extra:
            in_specs.append(pl.BlockSpec(v.shape, lambda i: (0, 0)))
    out_dim = w2.shape[1]

    def body(*refs):
        xr = refs[:nin]
        wr = refs[nin:2 * nin]
        b1r, w2r, b2r = refs[2 * nin:2 * nin + 3]
        o = refs[-1]
        acc = jnp.dot(xr[0][...], wr[0][...], preferred_element_type=jnp.float32)
        for k in range(1, nin):
            acc = acc + jnp.dot(xr[k][...], wr[k][...],
                                preferred_element_type=jnp.float32)
        h = jnp.maximum(acc + b1r[...], 0.0)
        y = jnp.dot(h, w2r[...], preferred_element_type=jnp.float32) + b2r[...]
        if ln:
            gr, br, mr = refs[2 * nin + 3:2 * nin + 6]
            m = mr[...]
            mu = jnp.dot(y, m, preferred_element_type=jnp.float32,
                         precision=lax.Precision.HIGHEST)
            yc = y - mu
            var = jnp.dot(yc * yc, m, preferred_element_type=jnp.float32,
                          precision=lax.Precision.HIGHEST)
            y = yc * lax.rsqrt(var + 1e-5) * gr[...] + br[...]
        o[...] = y

    args = [a for a, _ in inputs] + list(w1s) + [b1, w2, b2] + extra
    return pl.pallas_call(
        body,
        grid=(rows // block,),
        in_specs=in_specs,
        out_specs=pl.BlockSpec((block, out_dim), lambda i: (i, 0)),
        out_shape=jax.ShapeDtypeStruct((out_pad, out_dim), jnp.float32),
    )(*args)


def _node_update_tc(node, summ, cntm, sumc, cntc, p, block=1000):
    w1 = p["W1"]
    w1a = _bd(w1[0:HID])
    w1b = _bd(w1[HID:2 * HID])
    w1c = _bd(w1[2 * HID:3 * HID])
    b1 = _t2(p["b1"])
    w2 = _bd(p["W2"])
    b2 = _t2(p["b2"])
    g = _t2(p["g"])
    beta = _t2(p["beta"])
    m = _ln_m()

    def body(nr, smr, cmr, scr, ccr, w1ar, w1br, w1cr, b1r, w2r, b2r, gr, br,
             mr, o):
        aggm = smr[...] / jnp.maximum(cmr[...], 1.0)
        aggc = scr[...] / jnp.maximum(ccr[...], 1.0)
        acc = (jnp.dot(nr[...], w1ar[...], preferred_element_type=jnp.float32)
               + jnp.dot(aggm, w1br[...], preferred_element_type=jnp.float32)
               + jnp.dot(aggc, w1cr[...], preferred_element_type=jnp.float32))
        h = jnp.maximum(acc + b1r[...], 0.0)
        y = jnp.dot(h, w2r[...], preferred_element_type=jnp.float32) + b2r[...]
        mm = mr[...]
        mu = jnp.dot(y, mm, preferred_element_type=jnp.float32,
                     precision=lax.Precision.HIGHEST)
        yc = y - mu
        var = jnp.dot(yc * yc, mm, preferred_element_type=jnp.float32,
                      precision=lax.Precision.HIGHEST)
        o[...] = yc * lax.rsqrt(var + 1e-5) * gr[...] + br[...]

    full = lambda a: pl.BlockSpec(a.shape, lambda i: (0, 0))
    rowspec = pl.BlockSpec((block, 2 * HID), lambda i: (i, 0))
    np2 = N_NODES // 2
    return pl.pallas_call(
        body,
        grid=(np2 // block,),
        in_specs=[rowspec, rowspec, rowspec, rowspec, rowspec,
                  full(w1a), full(w1b), full(w1c), full(b1),
                  full(w2), full(b2), full(g), full(beta), full(m)],
        out_specs=rowspec,
        out_shape=jax.ShapeDtypeStruct((np2, 2 * HID), jnp.float32),
    )(node, summ, cntm, sumc, cntc, w1a, w1b, w1c, b1, w2, b2, g, beta, m)


def _sc_gather(table, idx):
    total = idx.shape[0]
    per_w = total // NW
    n_g = per_w // GRP
    n_g2 = n_g // 2
    mesh = plsc.VectorSubcoreMesh(core_axis_name="c", subcore_axis_name="s")

    @functools.partial(
        pl.kernel, mesh=mesh,
        out_type=jax.ShapeDtypeStruct((total, HID), jnp.float32),
        compiler_params=pltpu.CompilerParams(use_tc_tiling_on_sc=False),
        scratch_types=[
            pltpu.VMEM((2, GRP), jnp.int32),
            pltpu.VMEM((2, GRP, HID), jnp.float32),
        ] + [pltpu.SemaphoreType.DMA] * 6)
    def k(table_hbm, idx_hbm, out_hbm, idx2, rows2, si0, si1, sg0, sg1, so0, so1):
        c = lax.axis_index("c")
        s = lax.axis_index("s")
        base = (s * NC + c) * per_w
        si = (si0, si1)
        sg = (sg0, sg1)
        so = (so0, so1)
        pltpu.async_copy(idx_hbm.at[pl.ds(base, GRP)], idx2.at[0], si0)
        pltpu.async_copy(idx_hbm.at[pl.ds(base + GRP, GRP)], idx2.at[1], si1)

        def g2_body(g2, carry):
            for par in range(2):
                g = g2 * 2 + par
                off = base + g * GRP
                pltpu.make_async_copy(
                    idx_hbm.at[pl.ds(off, GRP)], idx2.at[par], si[par]).wait()

                @pl.when(g2 > 0)
                def _():
                    pltpu.make_async_copy(
                        rows2.at[par], out_hbm.at[pl.ds(off, GRP)],
                        so[par]).wait()

                for b in range(GK):
                    pltpu.async_copy(
                        table_hbm.at[idx2.at[par, pl.ds(b * SUB, SUB)]],
                        rows2.at[par, pl.ds(b * SUB, SUB)], sg[par])
                for b in range(GK):
                    pltpu.make_async_copy(
                        table_hbm.at[idx2.at[par, pl.ds(0, SUB)]],
                        rows2.at[par, pl.ds(0, SUB)], sg[par]).wait()
                pltpu.async_copy(rows2.at[par], out_hbm.at[pl.ds(off, GRP)],
                                 so[par])

                @pl.when(g2 + 1 < n_g2)
                def _():
                    pltpu.async_copy(
                        idx_hbm.at[pl.ds(off + 2 * GRP, GRP)], idx2.at[par],
                        si[par])
            return carry

        lax.fori_loop(0, n_g2, g2_body, 0)
        pltpu.make_async_copy(rows2.at[0], out_hbm.at[pl.ds(base, GRP)],
                              so0).wait()
        pltpu.make_async_copy(rows2.at[1], out_hbm.at[pl.ds(base, GRP)],
                              so1).wait()

    return k(table, idx)


def _sc_scatter_sum(vals_list, dst_list, d):
    nph = len(dst_list)
    count_mode = vals_list[0] is None
    mesh = plsc.VectorSubcoreMesh(core_axis_name="c", subcore_axis_name="s")

    def body(*refs):
        nin = nph if count_mode else 2 * nph
        ins = refs[:nin]
        outs = refs[nin:nin + nph]
        idx2, li2, zw, acc, rows = refs[nin + nph:nin + nph + 5]
        sems = refs[nin + nph + 5:]
        si = sems[0:2]
        ssc = sems[2:4]
        sz = sems[4]
        sr = sems[5:7]
        c = lax.axis_index("c")
        s = lax.axis_index("s")
        nbase = c * NHALF
        zero16 = (lax.iota(jnp.int32, 16) * 0).astype(jnp.float32)
        nz = (NZCH - 1 - s) // NS + 1

        def zrow(i, carry):
            for j in range(d // 16):
                zw[0, i, pl.ds(j * 16, 16)] = zero16
            return carry
        lax.fori_loop(0, WCH, zrow, 0)
        if count_mode:
            one16 = zero16 + 1.0

            def orow(i, carry):
                for j in range(d // 16):
                    rows[i, pl.ds(j * 16, 16)] = one16
                return carry
            lax.fori_loop(0, GRP_S, orow, 0)

        for ph in range(nph):
            if count_mode:
                dst_hbm = ins[ph]
                vals_hbm = None
            else:
                vals_hbm, dst_hbm = ins[2 * ph], ins[2 * ph + 1]
            out_hbm = outs[ph]
            e_pad = dst_hbm.shape[0]
            per_t = e_pad // NS
            n_g2 = per_t // GRP_S // 2

            def zfire(k2, carry):
                start = (s + k2 * NS) * WCH
                pltpu.async_copy(zw.at[0], acc.at[pl.ds(start, WCH)], sz)
                return carry
            lax.fori_loop(0, nz, zfire, 0)

            def zwait(k2, carry):
                pltpu.make_async_copy(zw.at[0], acc.at[pl.ds(0, WCH)],
                                      sz).wait()
                return carry
            lax.fori_loop(0, nz, zwait, 0)
            plsc.subcore_barrier()

            tbase = s * per_t
            pltpu.async_copy(dst_hbm.at[pl.ds(tbase, GRP_S)], idx2.at[0],
                             si[0])
            pltpu.async_copy(dst_hbm.at[pl.ds(tbase + GRP_S, GRP_S)],
                             idx2.at[1], si[1])
            if not count_mode:
                pltpu.async_copy(vals_hbm.at[pl.ds(tbase, GRP_S)], rows.at[0],
                                 sr[0])
                pltpu.async_copy(vals_hbm.at[pl.ds(tbase + GRP_S, GRP_S)],
                                 rows.at[1], sr[1])

            def g2_body(g2, carry):
                for par in range(2):
                    off = tbase + (g2 * 2 + par) * GRP_S
                    pltpu.make_async_copy(
                        dst_hbm.at[pl.ds(off, GRP_S)], idx2.at[par],
                        si[par]).wait()
                    if not count_mode:
                        pltpu.make_async_copy(
                            vals_hbm.at[pl.ds(off, GRP_S)], rows.at[par],
                            sr[par]).wait()
                    for j in range(GRP_S // 16):
                        v = idx2[par, pl.ds(j * 16, 16)]
                        li = v - nbase
                        ok = (li >= 0) & (li < NHALF)
                        li2[par, 0, pl.ds(j * 16, 16)] = jnp.where(ok, li,
                                                                   DUMP)
                    src = rows if count_mode else rows.at[par]
                    pltpu.async_copy(src, acc.at[li2.at[par, 0]], ssc[par],
                                     add=True)
                    pltpu.make_async_copy(src, acc.at[li2.at[par, 0]],
                                          ssc[par]).wait()

                    @pl.when(g2 + 1 < n_g2)
                    def _():
                        pltpu.async_copy(
                            dst_hbm.at[pl.ds(off + 2 * GRP_S, GRP_S)],
                            idx2.at[par], si[par])
                        if not count_mode:
                            pltpu.async_copy(
                                vals_hbm.at[pl.ds(off + 2 * GRP_S, GRP_S)],
                                rows.at[par], sr[par])
                return carry

            lax.fori_loop(0, n_g2, g2_body, 0)
            plsc.subcore_barrier()

            def wchunk(k2, carry):
                start = (s + k2 * NS) * WCH
                pltpu.sync_copy(acc.at[pl.ds(start, WCH)], zw.at[1])
                pltpu.sync_copy(zw.at[1], out_hbm.at[pl.ds(nbase + start,
                                                           WCH)])
                return carry
            lax.fori_loop(0, nz, wchunk, 0)
            plsc.subcore_barrier()

    scratch = [
        pltpu.VMEM((2, GRP_S), jnp.int32),
        pltpu.VMEM((2, 1, GRP_S), jnp.int32),
        pltpu.VMEM((2, WCH, d), jnp.float32),
        pltpu.VMEM_SHARED((ACC_ROWS, d), jnp.float32),
        pltpu.VMEM((GRP_S, d), jnp.float32) if count_mode
        else pltpu.VMEM((2, GRP_S, d), jnp.float32),
    ]
    scratch += [pltpu.SemaphoreType.DMA] * 7
    args = []
    for ph in range(nph):
        if not count_mode:
            args.append(vals_list[ph])
        args.append(dst_list[ph])
    return pl.kernel(
        body, mesh=mesh,
        out_type=tuple(
            jax.ShapeDtypeStruct((N_NODES, d), jnp.float32)
            for _ in range(nph)),
        compiler_params=pltpu.CompilerParams(use_tc_tiling_on_sc=False),
        scratch_types=scratch)(*args)


def _pad_rows(a, n):
    return jnp.pad(a, ((0, n - a.shape[0]),) + ((0, 0),) * (a.ndim - 1))


def _pad_k(w):
    return jnp.pad(w, ((0, 8 - w.shape[0]), (0, 0)))


def kernel(x, edge_attr, edge_contact_attr, edge_index, edge_contact_index, params):
    p = params
    s, r = edge_index[0], edge_index[1]
    cs, cr = edge_contact_index[0], edge_contact_index[1]

    scat_unit = NS * GRP_S * 2
    gath_unit = NW * GRP * 2
    em_pad = ((E_MESH + scat_unit - 1) // scat_unit) * scat_unit
    ec_pad = ((E_CONT + scat_unit - 1) // scat_unit) * scat_unit
    g_rows = E_MESH * 2 + E_CONT * 2
    g_pad = ((g_rows + gath_unit - 1) // gath_unit) * gath_unit

    r_pad = jnp.pad(r, (0, em_pad - E_MESH), constant_values=-1)
    cr_pad = jnp.pad(cr, (0, ec_pad - E_CONT), constant_values=-1)
    idx_all = jnp.concatenate(
        [s, r, cs, cr, jnp.zeros((g_pad - g_rows,), jnp.int32)])

    x16 = jnp.pad(x, ((0, 0), (0, 8 - x.shape[1]))).reshape(N_NODES // 2, 16)
    ea16 = jnp.pad(edge_attr, ((0, 0), (0, 8 - edge_attr.shape[1]))
                   ).reshape(E_MESH // 2, 16)
    ca16 = jnp.pad(edge_contact_attr,
                   ((0, 0), (0, 8 - edge_contact_attr.shape[1]))
                   ).reshape(E_CONT // 2, 16)

    ne = p["node_enc"]
    node = _mlp_tc([(x16, 0)], [_pad_k(ne["W1"])], ne["b1"], ne["W2"],
                   ne["b2"], ne["g"], ne["beta"], N_NODES // 2, 1000,
                   N_NODES // 2)
    ee = p["edge_enc"]
    ea = _mlp_tc([(ea16, 0)], [_pad_k(ee["W1"])], ee["b1"], ee["W2"],
                 ee["b2"], ee["g"], ee["beta"], E_MESH // 2, 3200, E_MESH // 2)
    ce = p["cont_enc"]
    eca = _mlp_tc([(ca16, 0)], [_pad_k(ce["W1"])], ce["b1"], ce["W2"],
                  ce["b2"], ce["g"], ce["beta"], E_CONT // 2, 2000,
                  E_CONT // 2)

    cntm, cntc = _sc_scatter_sum([None, None], [r_pad, cr_pad], HID)
    cntm = cntm.reshape(N_NODES // 2, 2 * HID)
    cntc = cntc.reshape(N_NODES // 2, 2 * HID)

    for ps in p["steps"]:
        g = _sc_gather(node.reshape(N_NODES, HID), idx_all)
        gp = g.reshape(g_pad // 2, 2 * HID)
        em = ps["edge_mesh"]
        w1 = em["W1"]
        ea = _mlp_tc([(gp, 0), (gp, E_MESH // 2), (ea, 0)],
                     [w1[0:HID], w1[HID:2 * HID], w1[2 * HID:3 * HID]],
                     em["b1"], em["W2"], em["b2"], em["g"], em["beta"],
                     E_MESH // 2, 3200, em_pad // 2)
        ecp = ps["edge_cont"]
        w1 = ecp["W1"]
        eca = _mlp_tc([(gp, E_MESH), (gp, E_MESH + E_CONT // 2), (eca, 0)],
                      [w1[0:HID], w1[HID:2 * HID], w1[2 * HID:3 * HID]],
                      ecp["b1"], ecp["W2"], ecp["b2"], ecp["g"], ecp["beta"],
                      E_CONT // 2, 2000, ec_pad // 2)
        summ, sumc = _sc_scatter_sum(
            [ea.reshape(em_pad, HID), eca.reshape(ec_pad, HID)],
            [r_pad, cr_pad], HID)
        node = _node_update_tc(node, summ.reshape(N_NODES // 2, 2 * HID),
                               cntm, sumc.reshape(N_NODES // 2, 2 * HID),
                               cntc, ps["node"])

    de = p["dec"]
    out = _mlp_tc([(node, 0)], [de["W1"]], de["b1"], de["W2"], de["b2"],
                  None, None, N_NODES // 2, 1000, N_NODES // 2)
    return out.reshape(N_NODES, 3)


# speedup vs baseline: 2.8083x; 1.0300x over previous
import functools

import jax
import jax.numpy as jnp
from jax import lax
from jax.experimental import pallas as pl
from jax.experimental.pallas import tpu as pltpu
from jax.experimental.pallas import tpu_sc as plsc

N_NODES = 50000
HID = 64
E_MESH = 800000
E_CONT = 100000

NC = 2
NS = 16
NW = NC * NS

SUB = 128
GK = 4
GRP = SUB * GK
GRP_S = SUB
NHALF = N_NODES // NC
ACC_ROWS = NHALF + 8
DUMP = NHALF
WCH = 100
NZCH = NHALF // WCH


def _bd(w):
    z = jnp.zeros_like(w)
    return jnp.concatenate(
        [jnp.concatenate([w, z], axis=1), jnp.concatenate([z, w], axis=1)],
        axis=0)


def _t2(v):
    return jnp.concatenate([v, v]).reshape(1, -1)


def _ln_m():
    e = jnp.eye(2, dtype=jnp.float32)
    return jnp.kron(e, jnp.full((HID, HID), 1.0 / HID, jnp.float32))


def _mlp_tc(inputs, w1s, b1, w2, b2, gamma, beta, rows, block, out_pad):
    nin = len(inputs)
    ln = gamma is not None
    b1 = _t2(b1)
    b2 = _t2(b2)
    w1s = [_bd(w) for w in w1s]
    w2 = _bd(w2)
    in_specs = []
    for arr, off in inputs:
        ob = off // block
        in_specs.append(
            pl.BlockSpec((block, arr.shape[1]), lambda i, ob=ob: (i + ob, 0)))
    for w in w1s:
        in_specs.append(pl.BlockSpec(w.shape, lambda i: (0, 0)))
    for v in (b1, w2, b2):
        in_specs.append(pl.BlockSpec(v.shape, lambda i: (0, 0)))
    extra = []
    if ln:
        extra = [_t2(gamma), _t2(beta), _ln_m()]
        for v in extra:
            in_specs.append(pl.BlockSpec(v.shape, lambda i: (0, 0)))
    out_dim = w2.shape[1]

    def body(*refs):
        xr = refs[:nin]
        wr = refs[nin:2 * nin]
        b1r, w2r, b2r = refs[2 * nin:2 * nin + 3]
        o = refs[-1]
        acc = jnp.dot(xr[0][...], wr[0][...], preferred_element_type=jnp.float32)
        for k in range(1, nin):
            acc = acc + jnp.dot(xr[k][...], wr[k][...],
                                preferred_element_type=jnp.float32)
        h = jnp.maximum(acc + b1r[...], 0.0)
        y = jnp.dot(h, w2r[...], preferred_element_type=jnp.float32) + b2r[...]
        if ln:
            gr, br, mr = refs[2 * nin + 3:2 * nin + 6]
            m = mr[...]
            mu = jnp.dot(y, m, preferred_element_type=jnp.float32,
                         precision=lax.Precision.HIGHEST)
            yc = y - mu
            var = jnp.dot(yc * yc, m, preferred_element_type=jnp.float32,
                          precision=lax.Precision.HIGHEST)
            y = yc * lax.rsqrt(var + 1e-5) * gr[...] + br[...]
        o[...] = y

    args = [a for a, _ in inputs] + list(w1s) + [b1, w2, b2] + extra
    return pl.pallas_call(
        body,
        grid=(rows // block,),
        in_specs=in_specs,
        out_specs=pl.BlockSpec((block, out_dim), lambda i: (i, 0)),
        out_shape=jax.ShapeDtypeStruct((out_pad, out_dim), jnp.float32),
    )(*args)


def _node_update_tc(node, summ, cntm, sumc, cntc, p, block=1000):
    w1 = p["W1"]
    w1a = _bd(w1[0:HID])
    w1b = _bd(w1[HID:2 * HID])
    w1c = _bd(w1[2 * HID:3 * HID])
    b1 = _t2(p["b1"])
    w2 = _bd(p["W2"])
    b2 = _t2(p["b2"])
    g = _t2(p["g"])
    beta = _t2(p["beta"])
    m = _ln_m()

    def body(nr, smr, cmr, scr, ccr, w1ar, w1br, w1cr, b1r, w2r, b2r, gr, br,
             mr, o):
        aggm = smr[...] / jnp.maximum(cmr[...], 1.0)
        aggc = scr[...] / jnp.maximum(ccr[...], 1.0)
        acc = (jnp.dot(nr[...], w1ar[...], preferred_element_type=jnp.float32)
               + jnp.dot(aggm, w1br[...], preferred_element_type=jnp.float32)
               + jnp.dot(aggc, w1cr[...], preferred_element_type=jnp.float32))
        h = jnp.maximum(acc + b1r[...], 0.0)
        y = jnp.dot(h, w2r[...], preferred_element_type=jnp.float32) + b2r[...]
        mm = mr[...]
        mu = jnp.dot(y, mm, preferred_element_type=jnp.float32,
                     precision=lax.Precision.HIGHEST)
        yc = y - mu
        var = jnp.dot(yc * yc, mm, preferred_element_type=jnp.float32,
                      precision=lax.Precision.HIGHEST)
        o[...] = yc * lax.rsqrt(var + 1e-5) * gr[...] + br[...]

    full = lambda a: pl.BlockSpec(a.shape, lambda i: (0, 0))
    rowspec = pl.BlockSpec((block, 2 * HID), lambda i: (i, 0))
    np2 = N_NODES // 2
    return pl.pallas_call(
        body,
        grid=(np2 // block,),
        in_specs=[rowspec, rowspec, rowspec, rowspec, rowspec,
                  full(w1a), full(w1b), full(w1c), full(b1),
                  full(w2), full(b2), full(g), full(beta), full(m)],
        out_specs=rowspec,
        out_shape=jax.ShapeDtypeStruct((np2, 2 * HID), jnp.float32),
    )(node, summ, cntm, sumc, cntc, w1a, w1b, w1c, b1, w2, b2, g, beta, m)


def _sc_gather(table, idx):
    total = idx.shape[0]
    per_w = total // NW
    n_g = per_w // GRP
    n_g2 = n_g // 2
    mesh = plsc.VectorSubcoreMesh(core_axis_name="c", subcore_axis_name="s")

    @functools.partial(
        pl.kernel, mesh=mesh,
        out_type=jax.ShapeDtypeStruct((total, HID), jnp.float32),
        compiler_params=pltpu.CompilerParams(use_tc_tiling_on_sc=False),
        scratch_types=[
            pltpu.VMEM((2, GRP), jnp.int32),
            pltpu.VMEM((2, GRP, HID), jnp.float32),
        ] + [pltpu.SemaphoreType.DMA] * 6)
    def k(table_hbm, idx_hbm, out_hbm, idx2, rows2, si0, si1, sg0, sg1, so0, so1):
        c = lax.axis_index("c")
        s = lax.axis_index("s")
        base = (s * NC + c) * per_w
        si = (si0, si1)
        sg = (sg0, sg1)
        so = (so0, so1)
        pltpu.async_copy(idx_hbm.at[pl.ds(base, GRP)], idx2.at[0], si0)
        pltpu.async_copy(idx_hbm.at[pl.ds(base + GRP, GRP)], idx2.at[1], si1)

        def g2_body(g2, carry):
            for par in range(2):
                g = g2 * 2 + par
                off = base + g * GRP
                pltpu.make_async_copy(
                    idx_hbm.at[pl.ds(off, GRP)], idx2.at[par], si[par]).wait()

                @pl.when(g2 > 0)
                def _():
                    pltpu.make_async_copy(
                        rows2.at[par], out_hbm.at[pl.ds(off, GRP)],
                        so[par]).wait()

                for b in range(GK):
                    pltpu.async_copy(
                        table_hbm.at[idx2.at[par, pl.ds(b * SUB, SUB)]],
                        rows2.at[par, pl.ds(b * SUB, SUB)], sg[par])
                for b in range(GK):
                    pltpu.make_async_copy(
                        table_hbm.at[idx2.at[par, pl.ds(0, SUB)]],
                        rows2.at[par, pl.ds(0, SUB)], sg[par]).wait()
                pltpu.async_copy(rows2.at[par], out_hbm.at[pl.ds(off, GRP)],
                                 so[par])

                @pl.when(g2 + 1 < n_g2)
                def _():
                    pltpu.async_copy(
                        idx_hbm.at[pl.ds(off + 2 * GRP, GRP)], idx2.at[par],
                        si[par])
            return carry

        lax.fori_loop(0, n_g2, g2_body, 0)
        pltpu.make_async_copy(rows2.at[0], out_hbm.at[pl.ds(base, GRP)],
                              so0).wait()
        pltpu.make_async_copy(rows2.at[1], out_hbm.at[pl.ds(base, GRP)],
                              so1).wait()

    return k(table, idx)


def _sc_scatter_sum(vals_list, dst_list, d):
    nph = len(dst_list)
    count_mode = vals_list[0] is None
    mesh = plsc.VectorSubcoreMesh(core_axis_name="c", subcore_axis_name="s")

    def body(*refs):
        nin = nph if count_mode else 2 * nph
        ins = refs[:nin]
        outs = refs[nin:nin + nph]
        idx2, li2, zw, acc, rows = refs[nin + nph:nin + nph + 5]
        sems = refs[nin + nph + 5:]
        si = sems[0:2]
        ssc = sems[2:4]
        sz = sems[4]
        sr = sems[5:7]
        c = lax.axis_index("c")
        s = lax.axis_index("s")
        nbase = c * NHALF
        zero16 = (lax.iota(jnp.int32, 16) * 0).astype(jnp.float32)
        nz = (NZCH - 1 - s) // NS + 1

        def zrow(i, carry):
            for j in range(d // 16):
                zw[0, i, pl.ds(j * 16, 16)] = zero16
            return carry
        lax.fori_loop(0, WCH, zrow, 0)
        if count_mode:
            one16 = zero16 + 1.0

            def orow(i, carry):
                for j in range(d // 16):
                    rows[i, pl.ds(j * 16, 16)] = one16
                return carry
            lax.fori_loop(0, GRP_S, orow, 0)

        for ph in range(nph):
            if count_mode:
                dst_hbm = ins[ph]
                vals_hbm = None
            else:
                vals_hbm, dst_hbm = ins[2 * ph], ins[2 * ph + 1]
            out_hbm = outs[ph]
            e_pad = dst_hbm.shape[0]
            per_t = e_pad // NS
            n_g2 = per_t // GRP_S // 2

            def zfire(k2, carry):
                start = (s + k2 * NS) * WCH
                pltpu.async_copy(zw.at[0], acc.at[pl.ds(start, WCH)], sz)
                return carry
            lax.fori_loop(0, nz, zfire, 0)

            def zwait(k2, carry):
                pltpu.make_async_copy(zw.at[0], acc.at[pl.ds(0, WCH)],
                                      sz).wait()
                return carry
            lax.fori_loop(0, nz, zwait, 0)
            plsc.subcore_barrier()

            tbase = s * per_t
            pltpu.async_copy(dst_hbm.at[pl.ds(tbase, GRP_S)], idx2.at[0],
                             si[0])
            pltpu.async_copy(dst_hbm.at[pl.ds(tbase + GRP_S, GRP_S)],
                             idx2.at[1], si[1])
            if not count_mode:
                pltpu.async_copy(vals_hbm.at[pl.ds(tbase, GRP_S)], rows.at[0],
                                 sr[0])
                pltpu.async_copy(vals_hbm.at[pl.ds(tbase + GRP_S, GRP_S)],
                                 rows.at[1], sr[1])

            def g2_body(g2, carry):
                for par in range(2):
                    off = tbase + (g2 * 2 + par) * GRP_S
                    pltpu.make_async_copy(
                        dst_hbm.at[pl.ds(off, GRP_S)], idx2.at[par],
                        si[par]).wait()
                    if not count_mode:
                        pltpu.make_async_copy(
                            vals_hbm.at[pl.ds(off, GRP_S)], rows.at[par],
                            sr[par]).wait()
                    for j in range(GRP_S // 16):
                        v = idx2[par, pl.ds(j * 16, 16)]
                        li = v - nbase
                        ok = (li >= 0) & (li < NHALF)
                        li2[par, 0, pl.ds(j * 16, 16)] = jnp.where(ok, li,
                                                                   DUMP)
                    src = rows if count_mode else rows.at[par]
                    pltpu.async_copy(src, acc.at[li2.at[par, 0]], ssc[par],
                                     add=True)
                    pltpu.make_async_copy(src, acc.at[li2.at[par, 0]],
                                          ssc[par]).wait()

                    @pl.when(g2 + 1 < n_g2)
                    def _():
                        pltpu.async_copy(
                            dst_hbm.at[pl.ds(off + 2 * GRP_S, GRP_S)],
                            idx2.at[par], si[par])
                        if not count_mode:
                            pltpu.async_copy(
                                vals_hbm.at[pl.ds(off + 2 * GRP_S, GRP_S)],
                                rows.at[par], sr[par])
                return carry

            lax.fori_loop(0, n_g2, g2_body, 0)
            plsc.subcore_barrier()

            def wchunk(k2, carry):
                start = (s + k2 * NS) * WCH
                pltpu.sync_copy(acc.at[pl.ds(start, WCH)], zw.at[1])
                pltpu.sync_copy(zw.at[1], out_hbm.at[pl.ds(nbase + start,
                                                           WCH)])
                return carry
            lax.fori_loop(0, nz, wchunk, 0)
            plsc.subcore_barrier()

    scratch = [
        pltpu.VMEM((2, GRP_S), jnp.int32),
        pltpu.VMEM((2, 1, GRP_S), jnp.int32),
        pltpu.VMEM((2, WCH, d), jnp.float32),
        pltpu.VMEM_SHARED((ACC_ROWS, d), jnp.float32),
        pltpu.VMEM((GRP_S, d), jnp.float32) if count_mode
        else pltpu.VMEM((2, GRP_S, d), jnp.float32),
    ]
    scratch += [pltpu.SemaphoreType.DMA] * 7
    args = []
    for ph in range(nph):
        if not count_mode:
            args.append(vals_list[ph])
        args.append(dst_list[ph])
    return pl.kernel(
        body, mesh=mesh,
        out_type=tuple(
            jax.ShapeDtypeStruct((N_NODES, d), jnp.float32)
            for _ in range(nph)),
        compiler_params=pltpu.CompilerParams(use_tc_tiling_on_sc=False),
        scratch_types=scratch)(*args)


def _pad_rows(a, n):
    return jnp.pad(a, ((0, n - a.shape[0]),) + ((0, 0),) * (a.ndim - 1))


def _pad_k(w):
    return jnp.pad(w, ((0, 8 - w.shape[0]), (0, 0)))


def kernel(x, edge_attr, edge_contact_attr, edge_index, edge_contact_index, params):
    p = params
    s, r = edge_index[0], edge_index[1]
    cs, cr = edge_contact_index[0], edge_contact_index[1]

    scat_unit = NS * GRP_S * 2
    gath_unit = NW * GRP * 2
    em_pad = ((E_MESH + scat_unit - 1) // scat_unit) * scat_unit
    ec_pad = ((E_CONT + scat_unit - 1) // scat_unit) * scat_unit
    g_rows = E_MESH * 2 + E_CONT * 2
    g_pad = ((g_rows + gath_unit - 1) // gath_unit) * gath_unit

    r_pad = jnp.pad(r, (0, em_pad - E_MESH), constant_values=-1)
    cr_pad = jnp.pad(cr, (0, ec_pad - E_CONT), constant_values=-1)
    idx_all = jnp.concatenate(
        [s, r, cs, cr, jnp.zeros((g_pad - g_rows,), jnp.int32)])

    x6 = x.reshape(N_NODES // 2, 6)
    ea12 = edge_attr.reshape(E_MESH // 2, 12)
    ca6 = edge_contact_attr.reshape(E_CONT // 2, 6)
    ne = p["node_enc"]
    node = _mlp_tc([(x6, 0)], [ne["W1"]], ne["b1"], ne["W2"], ne["b2"],
                   ne["g"], ne["beta"], N_NODES // 2, 1000, N_NODES // 2)
    ee = p["edge_enc"]
    ea = _mlp_tc([(ea12, 0)], [ee["W1"]], ee["b1"], ee["W2"], ee["b2"],
                 ee["g"], ee["beta"], E_MESH // 2, 3200, E_MESH // 2)
    ce = p["cont_enc"]
    eca = _mlp_tc([(ca6, 0)], [ce["W1"]], ce["b1"], ce["W2"], ce["b2"],
                  ce["g"], ce["beta"], E_CONT // 2, 2000, E_CONT // 2)

    cntm, cntc = _sc_scatter_sum([None, None], [r_pad, cr_pad], HID)
    cntm = cntm.reshape(N_NODES // 2, 2 * HID)
    cntc = cntc.reshape(N_NODES // 2, 2 * HID)

    for ps in p["steps"]:
        g = _sc_gather(node.reshape(N_NODES, HID), idx_all)
        gp = g.reshape(g_pad // 2, 2 * HID)
        em = ps["edge_mesh"]
        w1 = em["W1"]
        ea = _mlp_tc([(gp, 0), (gp, E_MESH // 2), (ea, 0)],
                     [w1[0:HID], w1[HID:2 * HID], w1[2 * HID:3 * HID]],
                     em["b1"], em["W2"], em["b2"], em["g"], em["beta"],
                     E_MESH // 2, 3200, em_pad // 2)
        ecp = ps["edge_cont"]
        w1 = ecp["W1"]
        eca = _mlp_tc([(gp, E_MESH), (gp, E_MESH + E_CONT // 2), (eca, 0)],
                      [w1[0:HID], w1[HID:2 * HID], w1[2 * HID:3 * HID]],
                      ecp["b1"], ecp["W2"], ecp["b2"], ecp["g"], ecp["beta"],
                      E_CONT // 2, 2000, ec_pad // 2)
        summ, sumc = _sc_scatter_sum(
            [ea.reshape(em_pad, HID), eca.reshape(ec_pad, HID)],
            [r_pad, cr_pad], HID)
        node = _node_update_tc(node, summ.reshape(N_NODES // 2, 2 * HID),
                               cntm, sumc.reshape(N_NODES // 2, 2 * HID),
                               cntc, ps["node"])

    de = p["dec"]
    out = _mlp_tc([(node, 0)], [de["W1"]], de["b1"], de["W2"], de["b2"],
                  None, None, N_NODES // 2, 1000, N_NODES // 2)
    return out.reshape(N_NODES, 3)
